# Initial kernel scaffold; baseline (speedup 1.0000x reference)
import jax
import jax.numpy as jnp
from jax import lax
from jax.experimental import pallas as pl
from jax.experimental.pallas import tpu as pltpu
from jax.experimental.pallas import tpu_sc as plsc

_B = 16384
_L = 50
_D = 128
_TOT = _B * _L
_NC = 2
_NS = 16
_NW = _NC * _NS
_PER_W = _TOT // _NW
_CHUNK = 128
_NCH = _PER_W // _CHUNK

_MM_BLK = 2000


def _proj_body(x_ref, w_ref, o_ref):
    o_ref[...] = lax.dot_general(
        x_ref[...], w_ref[...],
        (((1,), (1,)), ((), ())),
        preferred_element_type=jnp.float32,
    )


def _project_table(emb, W):
    m = emb.shape[0]
    grid = m // _MM_BLK
    return pl.pallas_call(
        _proj_body,
        grid=(grid,),
        in_specs=[
            pl.BlockSpec((_MM_BLK, _D), lambda i: (i, 0)),
            pl.BlockSpec((_D, _D), lambda i: (0, 0)),
        ],
        out_specs=pl.BlockSpec((_MM_BLK, _D), lambda i: (i, 0)),
        out_shape=jax.ShapeDtypeStruct((m, _D), jnp.float32),
    )(emb, W)


def _gather_body(tab_hbm, idx_hbm, out_hbm, idx_v, rows_a, rows_b, sem_a, sem_b):
    wid = lax.axis_index("s") * _NC + lax.axis_index("c")
    pltpu.sync_copy(idx_hbm.at[pl.ds(wid * _NCH, _NCH)], idx_v)
    out_base = wid * _PER_W

    pltpu.async_copy(tab_hbm.at[idx_v.at[0]], rows_a, sem_a)

    def step(j, carry):
        even = (j % 2) == 0

        @pl.when(j + 1 < _NCH)
        def _():
            @pl.when(even)
            def _():
                pltpu.async_copy(tab_hbm.at[idx_v.at[j + 1]], rows_b, sem_b)

            @pl.when(jnp.logical_not(even))
            def _():
                pltpu.async_copy(tab_hbm.at[idx_v.at[j + 1]], rows_a, sem_a)

        @pl.when(even)
        def _():
            pltpu.make_async_copy(tab_hbm.at[idx_v.at[0]], rows_a, sem_a).wait()
            pltpu.sync_copy(rows_a, out_hbm.at[pl.ds(out_base + j * _CHUNK, _CHUNK)])

        @pl.when(jnp.logical_not(even))
        def _():
            pltpu.make_async_copy(tab_hbm.at[idx_v.at[0]], rows_b, sem_b).wait()
            pltpu.sync_copy(rows_b, out_hbm.at[pl.ds(out_base + j * _CHUNK, _CHUNK)])

        return carry

    lax.fori_loop(0, _NCH, step, 0)


_gather = pl.kernel(
    _gather_body,
    out_type=jax.ShapeDtypeStruct((_TOT, _D), jnp.float32),
    mesh=plsc.VectorSubcoreMesh(core_axis_name="c", subcore_axis_name="s"),
    scratch_types=[
        pltpu.VMEM((_NCH, _CHUNK), jnp.int32),
        pltpu.VMEM((_CHUNK, _D), jnp.float32),
        pltpu.VMEM((_CHUNK, _D), jnp.float32),
        pltpu.SemaphoreType.DMA,
        pltpu.SemaphoreType.DMA,
    ],
)


@jax.jit
def kernel(idx, emb, W):
    proj = _project_table(emb, W)
    idx2d = idx.reshape(_NW * _NCH, _CHUNK).astype(jnp.int32)
    out = _gather(proj, idx2d)
    return out.reshape(_B, _L, _D)


# speedup vs baseline: 3.0065x; 3.0065x over previous
import jax
import jax.numpy as jnp
from jax import lax
from jax.experimental import pallas as pl
from jax.experimental.pallas import tpu as pltpu
from jax.experimental.pallas import tpu_sc as plsc

_B = 16384
_L = 50
_D = 128
_TOT = _B * _L
_NC = 2
_NS = 16
_NW = _NC * _NS
_PER_W = _TOT // _NW
_CHUNK = 128
_NCH = _PER_W // _CHUNK

_MM_BLK = 2000


def _proj_body(x_ref, w_ref, o_ref):
    o_ref[...] = lax.dot_general(
        x_ref[...], w_ref[...],
        (((1,), (1,)), ((), ())),
        preferred_element_type=jnp.float32,
    )


def _project_table(emb, W):
    m = emb.shape[0]
    grid = m // _MM_BLK
    return pl.pallas_call(
        _proj_body,
        grid=(grid,),
        in_specs=[
            pl.BlockSpec((_MM_BLK, _D), lambda i: (i, 0)),
            pl.BlockSpec((_D, _D), lambda i: (0, 0)),
        ],
        out_specs=pl.BlockSpec((_MM_BLK, _D), lambda i: (i, 0)),
        out_shape=jax.ShapeDtypeStruct((m, _D), jnp.float32),
    )(emb, W)


def _gather_body(tab_hbm, idx_hbm, out_hbm, idx_v, rows_a, rows_b, sem_a, sem_b):
    wid = lax.axis_index("s") * _NC + lax.axis_index("c")
    pltpu.sync_copy(idx_hbm.at[pl.ds(wid * _NCH, _NCH)], idx_v)
    out_base = wid * _PER_W

    pltpu.async_copy(tab_hbm.at[idx_v.at[0]], rows_a, sem_a)

    def step(j, carry):
        even = (j % 2) == 0

        @pl.when(j + 1 < _NCH)
        def _():
            @pl.when(even)
            def _():
                pltpu.async_copy(tab_hbm.at[idx_v.at[j + 1]], rows_b, sem_b)

            @pl.when(jnp.logical_not(even))
            def _():
                pltpu.async_copy(tab_hbm.at[idx_v.at[j + 1]], rows_a, sem_a)

        @pl.when(even)
        def _():
            pltpu.make_async_copy(tab_hbm.at[idx_v.at[0]], rows_a, sem_a).wait()
            pltpu.sync_copy(rows_a, out_hbm.at[pl.ds(out_base + j * _CHUNK, _CHUNK)])

        @pl.when(jnp.logical_not(even))
        def _():
            pltpu.make_async_copy(tab_hbm.at[idx_v.at[0]], rows_b, sem_b).wait()
            pltpu.sync_copy(rows_b, out_hbm.at[pl.ds(out_base + j * _CHUNK, _CHUNK)])

        return carry

    lax.fori_loop(0, _NCH, step, 0)


_gather = pl.kernel(
    _gather_body,
    out_type=jax.ShapeDtypeStruct((_TOT, _D), jnp.float32),
    mesh=plsc.VectorSubcoreMesh(
        core_axis_name="c", subcore_axis_name="s",
        num_cores=_NC, num_subcores=_NS,
    ),
    scratch_types=[
        pltpu.VMEM((_NCH, _CHUNK), jnp.int32),
        pltpu.VMEM((_CHUNK, _D), jnp.float32),
        pltpu.VMEM((_CHUNK, _D), jnp.float32),
        pltpu.SemaphoreType.DMA,
        pltpu.SemaphoreType.DMA,
    ],
)


@jax.jit
def kernel(idx, emb, W):
    proj = _project_table(emb, W)
    idx2d = idx.reshape(_NW * _NCH, _CHUNK).astype(jnp.int32)
    out = _gather(proj, idx2d)
    return out.reshape(_B, _L, _D)
